# Initial kernel scaffold; baseline (speedup 1.0000x reference)
import jax
import jax.numpy as jnp
from jax.experimental import pallas as pl


def kernel(tensor1, tensor2, W):
    raise NotImplementedError("write your pallas kernel here")


# speedup vs baseline: 2.0966x; 2.0966x over previous
import jax
import jax.numpy as jnp
from jax.experimental import pallas as pl

_TOKENS = 4096
_D = 1024
_E = 16
_BT = 64
_CHUNK = 128


def _gating_block(t1_ref, t2_ref, w_ref, out_ref):
    x = jnp.concatenate([t1_ref[...], t2_ref[...]], axis=1)
    cols = []
    for e in range(_E):
        w_row = w_ref[e, :]
        diff = w_row[None, :] - x
        sq = diff * diff
        acc = jnp.sum(sq[:, 0:_CHUNK], axis=1)
        for c in range(1, (2 * _D) // _CHUNK):
            acc = acc + jnp.sum(sq[:, c * _CHUNK:(c + 1) * _CHUNK], axis=1)
        cols.append(acc)
    d2 = jnp.stack(cols, axis=1)
    logits = -jnp.sqrt(d2)

    iota = jax.lax.broadcasted_iota(jnp.int32, logits.shape, 1)
    m1 = jnp.max(logits, axis=1, keepdims=True)
    i1 = jnp.min(jnp.where(logits == m1, iota, _E), axis=1, keepdims=True)
    sel1 = iota == i1
    masked = jnp.where(sel1, -jnp.inf, logits)
    m2 = jnp.max(masked, axis=1, keepdims=True)
    i2 = jnp.min(jnp.where(masked == m2, iota, _E), axis=1, keepdims=True)
    sel2 = iota == i2
    q = jnp.exp(m2 - m1)
    g1 = 1.0 / (1.0 + q)
    g2 = q / (1.0 + q)
    out_ref[...] = jnp.where(sel1, g1, 0.0) + jnp.where(sel2, g2, 0.0)


def kernel(tensor1, tensor2, W):
    grid = (_TOKENS // _BT,)
    return pl.pallas_call(
        _gating_block,
        grid=grid,
        in_specs=[
            pl.BlockSpec((_BT, _D), lambda i: (i, 0)),
            pl.BlockSpec((_BT, _D), lambda i: (i, 0)),
            pl.BlockSpec((_E, 2 * _D), lambda i: (0, 0)),
        ],
        out_specs=pl.BlockSpec((_BT, _E), lambda i: (i, 0)),
        out_shape=jax.ShapeDtypeStruct((_TOKENS, _E), jnp.float32),
    )(tensor1, tensor2, W)


# speedup vs baseline: 2.3423x; 1.1172x over previous
import jax
import jax.numpy as jnp
from jax.experimental import pallas as pl

_TOKENS = 4096
_D = 1024
_E = 16
_BT = 128
_CHUNK = 128


def _gating_block(t1_ref, t2_ref, w_ref, out_ref):
    d2_rows = []
    for tg in range(_BT // 8):
        r0 = tg * 8
        accs = [None] * _E
        for c in range((2 * _D) // _CHUNK):
            if c < _D // _CHUNK:
                xc = t1_ref[r0:r0 + 8, c * _CHUNK:(c + 1) * _CHUNK]
            else:
                cc = c - _D // _CHUNK
                xc = t2_ref[r0:r0 + 8, cc * _CHUNK:(cc + 1) * _CHUNK]
            for e in range(_E):
                w_row = w_ref[e, c * _CHUNK:(c + 1) * _CHUNK]
                diff = w_row[None, :] - xc
                sq = diff * diff
                p = jnp.sum(sq, axis=1, keepdims=True)
                accs[e] = p if c == 0 else accs[e] + p
        d2_rows.append(jnp.concatenate(accs, axis=1))
    d2 = jnp.concatenate(d2_rows, axis=0)
    logits = -jnp.sqrt(d2)

    iota = jax.lax.broadcasted_iota(jnp.int32, logits.shape, 1)
    m1 = jnp.max(logits, axis=1, keepdims=True)
    i1 = jnp.min(jnp.where(logits == m1, iota, _E), axis=1, keepdims=True)
    sel1 = iota == i1
    masked = jnp.where(sel1, -jnp.inf, logits)
    m2 = jnp.max(masked, axis=1, keepdims=True)
    i2 = jnp.min(jnp.where(masked == m2, iota, _E), axis=1, keepdims=True)
    sel2 = iota == i2
    q = jnp.exp(m2 - m1)
    g1 = 1.0 / (1.0 + q)
    g2 = q / (1.0 + q)
    out_ref[...] = jnp.where(sel1, g1, 0.0) + jnp.where(sel2, g2, 0.0)


def kernel(tensor1, tensor2, W):
    grid = (_TOKENS // _BT,)
    return pl.pallas_call(
        _gating_block,
        grid=grid,
        in_specs=[
            pl.BlockSpec((_BT, _D), lambda i: (i, 0)),
            pl.BlockSpec((_BT, _D), lambda i: (i, 0)),
            pl.BlockSpec((_E, 2 * _D), lambda i: (0, 0)),
        ],
        out_specs=pl.BlockSpec((_BT, _E), lambda i: (i, 0)),
        out_shape=jax.ShapeDtypeStruct((_TOKENS, _E), jnp.float32),
    )(tensor1, tensor2, W)
